# baseline (speedup 1.0000x reference)
import functools

import jax
import jax.numpy as jnp
from jax import lax
from jax.experimental import pallas as pl
from jax.experimental.pallas import tpu as pltpu

DIM = 128
NB = 128
BLK = 512
ATOM_PAD = 50176
MOTIF_PAD = 10240


def _mlp_relu(x, w1, b1, w2, b2):
    h = jnp.maximum(
        jnp.dot(x, w1, preferred_element_type=jnp.float32) + b1, 0.0)
    o = jnp.dot(h, w2, preferred_element_type=jnp.float32) + b2
    return jnp.maximum(o, 0.0)


def _gine_mlp_kernel(x_ref, agg_ref, w1_ref, b1_ref, w2_ref, b2_ref, o_ref):
    o_ref[...] = _mlp_relu(x_ref[...] + agg_ref[...], w1_ref[...],
                           b1_ref[...], w2_ref[...], b2_ref[...])


def _full(shape):
    return pl.BlockSpec(shape, lambda i: (0,) * len(shape))


def _rows(n_out=DIM):
    return pl.BlockSpec((BLK, n_out), lambda i: (i, 0))


@functools.partial(jax.jit, static_argnames=("npad",))
def gine_mlp(x, agg, p, npad):
    w1, b1, w2, b2 = p
    return pl.pallas_call(
        _gine_mlp_kernel,
        grid=(npad // BLK,),
        in_specs=[_rows(), _rows(), _full((DIM, DIM)), _full((1, DIM)),
                  _full((DIM, DIM)), _full((1, DIM))],
        out_specs=_rows(),
        out_shape=jax.ShapeDtypeStruct((npad, DIM), jnp.float32),
    )(x, agg, w1, b1.reshape(1, DIM), w2, b2.reshape(1, DIM))


def _atom_out_kernel(h1_ref, h2_ref, wc1_ref, wc2_ref, bc_ref,
                     x_ref, agg_ref, w1_ref, b1_ref, w2_ref, b2_ref, o_ref):
    cat = (jnp.dot(h1_ref[...], wc1_ref[...], preferred_element_type=jnp.float32)
           + jnp.dot(h2_ref[...], wc2_ref[...], preferred_element_type=jnp.float32)
           + bc_ref[...])
    a2a = jnp.maximum(cat, 0.0)
    m2a = _mlp_relu(x_ref[...] + agg_ref[...], w1_ref[...], b1_ref[...],
                    w2_ref[...], b2_ref[...])
    o_ref[...] = a2a + m2a


@jax.jit
def atom_out(h1, h2, cat_p, x, agg_m2a, m2a_p):
    wc, bc = cat_p
    w1, b1, w2, b2 = m2a_p
    return pl.pallas_call(
        _atom_out_kernel,
        grid=(ATOM_PAD // BLK,),
        in_specs=[_rows(), _rows(), _full((DIM, DIM)), _full((DIM, DIM)),
                  _full((1, DIM)), _rows(), _rows(), _full((DIM, DIM)),
                  _full((1, DIM)), _full((DIM, DIM)), _full((1, DIM))],
        out_specs=_rows(),
        out_shape=jax.ShapeDtypeStruct((ATOM_PAD, DIM), jnp.float32),
    )(h1, h2, wc[:DIM], wc[DIM:], bc.reshape(1, DIM), x, agg_m2a,
      w1, b1.reshape(1, DIM), w2, b2.reshape(1, DIM))


def _motif_out_kernel(x_ref, agg1_ref, w11, b11, w12, b12,
                      agg2_ref, w21, b21, w22, b22, o_ref):
    t1 = _mlp_relu(x_ref[...] + agg1_ref[...], w11[...], b11[...],
                   w12[...], b12[...])
    t2 = _mlp_relu(x_ref[...] + agg2_ref[...], w21[...], b21[...],
                   w22[...], b22[...])
    o_ref[...] = t1 + t2


@jax.jit
def motif_out(x, agg_a2m, a2m_p, agg_m2m, m2m_p):
    w11, b11, w12, b12 = a2m_p
    w21, b21, w22, b22 = m2m_p
    return pl.pallas_call(
        _motif_out_kernel,
        grid=(MOTIF_PAD // BLK,),
        in_specs=[_rows(), _rows(), _full((DIM, DIM)), _full((1, DIM)),
                  _full((DIM, DIM)), _full((1, DIM)), _rows(),
                  _full((DIM, DIM)), _full((1, DIM)), _full((DIM, DIM)),
                  _full((1, DIM))],
        out_specs=_rows(),
        out_shape=jax.ShapeDtypeStruct((MOTIF_PAD, DIM), jnp.float32),
    )(x, agg_a2m, w11, b11.reshape(1, DIM), w12, b12.reshape(1, DIM),
      agg_m2m, w21, b21.reshape(1, DIM), w22, b22.reshape(1, DIM))


def _pool_kernel(b_ref, x_ref, o_ref):
    @pl.when(pl.program_id(0) == 0)
    def _init():
        o_ref[...] = jnp.zeros_like(o_ref)

    ids = b_ref[0, 0, :]
    oneh = (ids[:, None]
            == lax.broadcasted_iota(jnp.int32, (BLK, NB), 1)).astype(jnp.float32)
    o_ref[...] += lax.dot_general(
        oneh, x_ref[...], (((0,), (0,)), ((), ())),
        preferred_element_type=jnp.float32, precision=lax.Precision.HIGHEST)


@functools.partial(jax.jit, static_argnames=("npad", "dout"))
def pool(batch3, x, npad, dout):
    return pl.pallas_call(
        _pool_kernel,
        grid=(npad // BLK,),
        in_specs=[pl.BlockSpec((1, 1, BLK), lambda i: (i, 0, 0)),
                  _rows(dout)],
        out_specs=_full((NB, dout)),
        out_shape=jax.ShapeDtypeStruct((NB, dout), jnp.float32),
    )(batch3, x)


def _aggregate(x_src, ei, ea, npad_dst):
    msg = jnp.maximum(x_src[ei[0]] + ea, 0.0)
    return jnp.zeros((npad_dst, DIM), jnp.float32).at[ei[1]].add(msg)


def kernel(x_atom, x_motif, ei_a2a, ei_a2m, ei_m2a, ei_m2m,
           ea_a2a, ea_a2m, ea_m2a, ea_m2m, batch_atom, batch_motif, params):
    na, nm = x_atom.shape[0], x_motif.shape[0]
    xa = jnp.zeros((ATOM_PAD, DIM), jnp.float32).at[:na].set(x_atom)
    xm = jnp.zeros((MOTIF_PAD, DIM), jnp.float32).at[:nm].set(x_motif)

    ba = jnp.full((ATOM_PAD,), NB, jnp.int32).at[:na].set(batch_atom)
    bm = jnp.full((MOTIF_PAD,), NB, jnp.int32).at[:nm].set(batch_motif)
    ba3 = ba.reshape(ATOM_PAD // BLK, 1, BLK)
    bm3 = bm.reshape(MOTIF_PAD // BLK, 1, BLK)

    xs_a, xs_m = [], []
    for lp in params['layers']:
        agg0 = _aggregate(xa, ei_a2a, ea_a2a, ATOM_PAD)
        h1 = gine_mlp(xa, agg0, lp['a2a']['l0'], ATOM_PAD)
        agg1 = _aggregate(h1, ei_a2a, ea_a2a, ATOM_PAD)
        h2 = gine_mlp(h1, agg1, lp['a2a']['l1'], ATOM_PAD)
        agg_m2a = _aggregate(xm, ei_m2a, ea_m2a, ATOM_PAD)
        oa = atom_out(h1, h2, lp['a2a']['cat'], xa, agg_m2a, lp['m2a'])
        agg_a2m = _aggregate(xa, ei_a2m, ea_a2m, MOTIF_PAD)
        agg_m2m = _aggregate(xm, ei_m2m, ea_m2m, MOTIF_PAD)
        om = motif_out(xm, agg_a2m, lp['a2m'], agg_m2m, lp['m2m'])
        xa, xm = oa, om
        xs_a.append(xa)
        xs_m.append(xm)

    ja = jnp.concatenate(xs_a, axis=1)
    jm = jnp.concatenate(xs_m, axis=1)
    pa = pool(ba3, ja, ATOM_PAD, 3 * DIM)
    pm = pool(bm3, jm, MOTIF_PAD, 3 * DIM)
    return (pa, pm)
